# baseline (speedup 1.0000x reference)
import functools

import jax
import jax.numpy as jnp
from jax import lax
from jax.experimental import pallas as pl
from jax.experimental.pallas import tpu as pltpu
from jax.experimental.pallas import tpu_sc as plsc

B, Kp, C, M = 2, 256, 512, 8
Kc = Kp * M
NBLK = 4
PB = Kp // NBLK
RB = PB * M
GRID = B * NBLK
PHI_M2 = 1.6 ** (-2)
JITTER = 1e-4
BETA = 0.5
K_TARGET = 1024.0
F32 = jnp.float32

NC, NS, L = 2, 16, 16
NW = NC * NS
RPW = (B * Kp) // NW
NCHUNK = Kp // L


def _sigmoid(x):
    return 1.0 / (1.0 + jnp.exp(-x))


def _softplus(x):
    return jnp.maximum(x, 0.0) + jnp.log(1.0 + jnp.exp(-jnp.abs(x)))


def _ln(x, g, b, eps=1e-5):
    m = jnp.mean(x, axis=-1, keepdims=True)
    v = jnp.mean((x - m) ** 2, axis=-1, keepdims=True)
    return (x - m) / jnp.sqrt(v + eps) * g + b


def _inv6(s00, s01, s02, s11, s12, s22):
    c00 = s11 * s22 - s12 * s12
    c01 = s02 * s12 - s01 * s22
    c02 = s01 * s12 - s02 * s11
    c11 = s00 * s22 - s02 * s02
    c12 = s01 * s02 - s00 * s12
    c22 = s00 * s11 - s01 * s01
    rdet = 1.0 / (s00 * c00 + s01 * c01 + s02 * c02)
    return c00 * rdet, c01 * rdet, c02 * rdet, c11 * rdet, c12 * rdet, c22 * rdet


def _quad(p00, p01, p02, p11, p12, p22, dx, dy, dz):
    return (p00 * dx * dx + p11 * dy * dy + p22 * dz * dz
            + 2.0 * (p01 * dx * dy + p02 * dx * dz + p12 * dy * dz))


def _sc_body(muT_hbm, sigT_hbm, sp_hbm, n_out, snei_out,
             mvT, sgT, idxv, rows, nbuf, sem):
    wid = lax.axis_index("s") * NC + lax.axis_index("c")
    bno = wid // NS
    jbase = (wid - bno * NS) * RPW

    pltpu.sync_copy(muT_hbm.at[pl.ds(bno * 3, 3), :], mvT)
    pltpu.sync_copy(sigT_hbm.at[pl.ds(bno * 9, 9), :], sgT)

    lane = lax.iota(jnp.int32, L)
    nloc_v = jnp.zeros((L,), jnp.int32)

    for r in range(RPW):
        jloc = jbase + r
        sel = lane == r
        mjx = jnp.sum(jnp.where(sel, mvT[0, pl.ds(jbase, L)], 0.0))
        mjy = jnp.sum(jnp.where(sel, mvT[1, pl.ds(jbase, L)], 0.0))
        mjz = jnp.sum(jnp.where(sel, mvT[2, pl.ds(jbase, L)], 0.0))

        def chunk_body(c, carry, mjx=mjx, mjy=mjy, mjz=mjz, jloc=jloc):
            mx, mi = carry
            off = c * L
            s00 = sgT[0, pl.ds(off, L)]
            s01 = sgT[1, pl.ds(off, L)]
            s02 = sgT[2, pl.ds(off, L)]
            s11 = sgT[4, pl.ds(off, L)]
            s12 = sgT[5, pl.ds(off, L)]
            s22 = sgT[8, pl.ds(off, L)]
            p00, p01, p02, p11, p12, p22 = _inv6(s00, s01, s02, s11, s12, s22)
            dx = mjx - mvT[0, pl.ds(off, L)]
            dy = mjy - mvT[1, pl.ds(off, L)]
            dz = mjz - mvT[2, pl.ds(off, L)]
            S = -0.5 * _quad(p00, p01, p02, p11, p12, p22, dx, dy, dz)
            kidx = lane + off
            S = jnp.where(kidx == jloc, -1e30, S)
            cond = S > mx
            mx = jnp.where(cond, S, mx)
            mi = jnp.where(cond, kidx.astype(F32), mi)
            return mx, mi

        mx, mi = lax.fori_loop(
            0, NCHUNK, chunk_body,
            (jnp.full((L,), -3e38, F32), jnp.zeros((L,), F32)))
        mbest = jnp.max(mx)
        nf = jnp.min(jnp.where(mx == mbest, mi, 1e9))
        nloc_v = jnp.where(sel, nf.astype(jnp.int32), nloc_v)

    nbuf[...] = nloc_v
    pltpu.sync_copy(nbuf, n_out.at[pl.ds(wid * RPW, RPW)])
    idxv[...] = nloc_v + bno * Kp
    pltpu.async_copy(sp_hbm.at[idxv], rows, sem).wait()
    pltpu.sync_copy(rows, snei_out.at[pl.ds(wid * RPW, RPW), :])


def _sc_retrieve(muT, sigT9, sp):
    mesh = plsc.VectorSubcoreMesh(core_axis_name="c", subcore_axis_name="s",
                                  num_cores=NC, num_subcores=NS)
    kern = pl.kernel(
        _sc_body,
        out_type=[jax.ShapeDtypeStruct((B * Kp,), jnp.int32),
                  jax.ShapeDtypeStruct((B * Kp, C), F32)],
        mesh=mesh,
        scratch_types=[pltpu.VMEM((3, Kp), F32),
                       pltpu.VMEM((9, Kp), F32),
                       pltpu.VMEM((RPW,), jnp.int32),
                       pltpu.VMEM((RPW, C), F32),
                       pltpu.VMEM((RPW,), jnp.int32),
                       pltpu.SemaphoreType.DMA],
        compiler_params=pltpu.CompilerParams(use_tc_tiling_on_sc=False,
                                             needs_layout_passes=False),
    )
    return kern(muT.reshape(B * 3, Kp), sigT9.reshape(B * 9, Kp), sp)


def _body(spb, mub, muall, sig9b, sig9all, maskb, xib, emb, nblk, sneib,
          glg, glb, gW1, gb1, gW2c, gb2,
          plg, plb, pW1, pb1, pW2c, pb2, gmW, gmb,
          out_s, out_mu, out_sig, out_g, out_gsum, out_loss):
    gid = pl.program_id(0)
    bno = gid // NBLK

    spb = spb[...]
    mub = mub[...]
    muall = muall[...]
    sig9b = sig9b[...]
    sig9all = sig9all[...]
    maskb = maskb[...]
    xib = xib[...]
    emb = emb[...]
    nblk = nblk[...]
    sneib = sneib[...]
    glg = glg[...]
    glb = glb[...]
    gW1 = gW1[...]
    gb1 = gb1[...]
    gW2c = gW2c[...]
    gb2 = gb2[...]
    plg = plg[...]
    plb = plb[...]
    pW1 = pW1[...]
    pb1 = pb1[...]
    pW2c = pW2c[...]
    pb2 = pb2[...]
    gmW = gmW[...]
    gmb = gmb[...]

    colk_i = jax.lax.broadcasted_iota(jnp.int32, (PB, Kp), 1)
    onehot = jnp.where(colk_i == nblk, 1.0, 0.0).astype(F32)

    hi = jax.lax.Precision.HIGHEST
    mu_nei = jnp.dot(onehot, muall, preferred_element_type=F32, precision=hi)
    sig_nei = jnp.dot(onehot, sig9all, preferred_element_type=F32, precision=hi)
    s_nei = sneib

    sb = [sig9b[:, i:i + 1] for i in (0, 1, 2, 4, 5, 8)]
    Pj = _inv6(*sb)
    sn = [sig_nei[:, i:i + 1] for i in (0, 1, 2, 4, 5, 8)]
    Pn = _inv6(*sn)

    a00 = sb[0] + 1e-6
    a11 = sb[3] + 1e-6
    a22 = sb[5] + 1e-6
    l11 = jnp.sqrt(a00)
    l21 = sb[1] / l11
    l31 = sb[2] / l11
    l22 = jnp.sqrt(a11 - l21 * l21)
    l32 = (sb[4] - l31 * l21) / l22
    l33 = jnp.sqrt(a22 - l31 * l31 - l32 * l32)

    hp = _ln(spb, plg, plb)
    hp = jnp.dot(hp, pW1, preferred_element_type=F32) + pb1
    hp = hp * _sigmoid(hp)
    apre = jnp.dot(hp, pW2c, preferred_element_type=F32) + pb2
    log_a = jnp.log(_softplus(apre) + 1e-8)

    r8 = jax.lax.broadcasted_iota(jnp.int32, (RB, PB), 0) // M
    cE = jax.lax.broadcasted_iota(jnp.int32, (RB, PB), 1)
    E8 = jnp.where(r8 == cE, 1.0, 0.0).astype(F32)
    rT = jax.lax.broadcasted_iota(jnp.int32, (RB, M), 0) % M
    cT = jax.lax.broadcasted_iota(jnp.int32, (RB, M), 1)
    T8 = jnp.where(rT == cT, 1.0, 0.0).astype(F32)

    def exp8(v):
        return jnp.dot(E8, v, preferred_element_type=F32, precision=hi)

    X = (spb[:, None, :] + emb[None, :, :]).reshape(RB, C)
    h = _ln(X, glg, glb)
    h = jnp.dot(h, gW1, preferred_element_type=F32) + gb1
    h = h * _sigmoid(h)
    blog = jnp.dot(h, gW2c, preferred_element_type=F32) + gb2
    blog = blog + BETA * exp8(log_a)
    g = _sigmoid(blog) * exp8(maskb)

    xi_bias = jnp.dot(emb, gmW, preferred_element_type=F32, precision=hi) + gmb
    xi = xib + jnp.dot(T8, xi_bias, preferred_element_type=F32, precision=hi)
    x1 = xi[:, 0:1]
    x2 = xi[:, 1:2]
    x3 = xi[:, 2:3]
    Le = exp8(jnp.concatenate([l11, l21, l22, l31, l32, l33], axis=1))
    mue = exp8(mub)
    mu0x = mue[:, 0:1] + Le[:, 0:1] * x1
    mu0y = mue[:, 1:2] + Le[:, 1:2] * x1 + Le[:, 2:3] * x2
    mu0z = mue[:, 2:3] + Le[:, 3:4] * x1 + Le[:, 4:5] * x2 + Le[:, 5:6] * x3

    Pje = exp8(jnp.concatenate(Pj, axis=1))
    Pne = exp8(jnp.concatenate(Pn, axis=1))
    mune = exp8(mu_nei)
    d0x = mu0x - mue[:, 0:1]
    d0y = mu0y - mue[:, 1:2]
    d0z = mu0z - mue[:, 2:3]
    d1x = mu0x - mune[:, 0:1]
    d1y = mu0y - mune[:, 1:2]
    d1z = mu0z - mune[:, 2:3]
    sc0 = -0.5 * _quad(Pje[:, 0:1], Pje[:, 1:2], Pje[:, 2:3],
                       Pje[:, 3:4], Pje[:, 4:5], Pje[:, 5:6], d0x, d0y, d0z)
    sc1 = -0.5 * _quad(Pne[:, 0:1], Pne[:, 1:2], Pne[:, 2:3],
                       Pne[:, 3:4], Pne[:, 4:5], Pne[:, 5:6], d1x, d1y, d1z)
    w0 = _sigmoid(sc0 - sc1)
    w1 = 1.0 - w0

    mcx = w0 * mue[:, 0:1] + w1 * mune[:, 0:1]
    mcy = w0 * mue[:, 1:2] + w1 * mune[:, 1:2]
    mcz = w0 * mue[:, 2:3] + w1 * mune[:, 2:3]
    out_mu[:, 0:1] = mcx
    out_mu[:, 1:2] = mcy
    out_mu[:, 2:3] = mcz

    sig6e = exp8(jnp.concatenate(sb, axis=1))
    sig6ne = exp8(jnp.concatenate(sn, axis=1))
    djx = mue[:, 0:1] - mcx
    djy = mue[:, 1:2] - mcy
    djz = mue[:, 2:3] - mcz
    dnx = mune[:, 0:1] - mcx
    dny = mune[:, 1:2] - mcy
    dnz = mune[:, 2:3] - mcz
    intra = (w0 * sig6e + w1 * sig6ne) * PHI_M2
    o00 = intra[:, 0:1] + w0 * djx * djx + w1 * dnx * dnx + JITTER
    o01 = intra[:, 1:2] + w0 * djx * djy + w1 * dnx * dny
    o02 = intra[:, 2:3] + w0 * djx * djz + w1 * dnx * dnz
    o11 = intra[:, 3:4] + w0 * djy * djy + w1 * dny * dny + JITTER
    o12 = intra[:, 4:5] + w0 * djy * djz + w1 * dny * dnz
    o22 = intra[:, 5:6] + w0 * djz * djz + w1 * dnz * dnz + JITTER
    out_sig[:, 0:1] = o00
    out_sig[:, 1:2] = o01
    out_sig[:, 2:3] = o02
    out_sig[:, 3:4] = o01
    out_sig[:, 4:5] = o11
    out_sig[:, 5:6] = o12
    out_sig[:, 6:7] = o02
    out_sig[:, 7:8] = o12
    out_sig[:, 8:9] = o22

    smix = w0 * exp8(spb) + w1 * exp8(s_nei)
    out_s[...] = g * smix
    out_g[...] = g

    @pl.when(gid == 0)
    def _init():
        out_gsum[...] = jnp.zeros((B, 1), F32)

    rowb = jax.lax.broadcasted_iota(jnp.int32, (B, 1), 0)
    out_gsum[...] += jnp.where(rowb == bno, jnp.sum(g), 0.0)

    @pl.when(gid == GRID - 1)
    def _loss():
        d = out_gsum[...] - K_TARGET
        out_loss[...] = jnp.sum(d * d, axis=0, keepdims=True) * (1.0 / B)


def kernel(s_parent, mu_p, Sigma_p, mask_parent, xi_noise, child_embed,
           gate_ln_g, gate_ln_b, gate_W1, gate_b1, gate_W2, gate_b2,
           prior_ln_g, prior_ln_b, prior_W1, prior_b1, prior_W2, prior_b2,
           geom_W, geom_b):
    f = F32
    sp = s_parent.reshape(B * Kp, C)
    mu = mu_p.reshape(B * Kp, 3)
    sig9 = Sigma_p.reshape(B * Kp, 9)
    muT = jnp.transpose(mu_p, (0, 2, 1))
    sigT9 = jnp.transpose(Sigma_p.reshape(B, Kp, 9), (0, 2, 1))
    maskf = mask_parent.reshape(B * Kp, 1)
    xif = xi_noise.reshape(B * Kc, 3)

    n_idx, s_nei = _sc_retrieve(muT, sigT9, sp)
    n2d = n_idx.reshape(B * Kp, 1)

    def bs(shape, idx):
        return pl.BlockSpec(shape, idx)

    blkmap = lambda g: (g, 0)
    batmap = lambda g: (g // NBLK, 0)
    full = lambda g: (0, 0)

    in_specs = [
        bs((PB, C), blkmap),
        bs((PB, 3), blkmap),
        bs((Kp, 3), batmap),
        bs((PB, 9), blkmap),
        bs((Kp, 9), batmap),
        bs((PB, 1), blkmap),
        bs((RB, 3), blkmap),
        bs((M, C), full),
        bs((PB, 1), blkmap),
        bs((PB, C), blkmap),
        bs((1, C), full), bs((1, C), full), bs((C, C), full), bs((1, C), full),
        bs((C, 1), full), bs((1, 1), full),
        bs((1, C), full), bs((1, C), full), bs((C, C), full), bs((1, C), full),
        bs((C, 1), full), bs((1, 1), full),
        bs((C, 3), full), bs((1, 3), full),
    ]
    out_specs = [
        bs((RB, C), blkmap),
        bs((RB, 3), blkmap),
        bs((RB, 9), blkmap),
        bs((RB, 1), blkmap),
        bs((B, 1), full),
        bs((1, 1), full),
    ]
    out_shapes = [
        jax.ShapeDtypeStruct((B * Kc, C), f),
        jax.ShapeDtypeStruct((B * Kc, 3), f),
        jax.ShapeDtypeStruct((B * Kc, 9), f),
        jax.ShapeDtypeStruct((B * Kc, 1), f),
        jax.ShapeDtypeStruct((B, 1), f),
        jax.ShapeDtypeStruct((1, 1), f),
    ]

    outs = pl.pallas_call(
        _body,
        grid=(GRID,),
        in_specs=in_specs,
        out_specs=out_specs,
        out_shape=out_shapes,
    )(sp, mu, mu, sig9, sig9, maskf, xif, child_embed, n2d, s_nei,
      gate_ln_g.reshape(1, C), gate_ln_b.reshape(1, C), gate_W1,
      gate_b1.reshape(1, C), gate_W2, gate_b2.reshape(1, 1),
      prior_ln_g.reshape(1, C), prior_ln_b.reshape(1, C), prior_W1,
      prior_b1.reshape(1, C), prior_W2, prior_b2.reshape(1, 1),
      geom_W, geom_b.reshape(1, 3))

    s_child, mu_child, sig9o, gout, _gsum, loss = outs
    return (s_child.reshape(B, Kc, C),
            mu_child.reshape(B, Kc, 3),
            sig9o.reshape(B, Kc, 3, 3),
            gout.reshape(B, Kc),
            loss.reshape(()))


# speedup vs baseline: 1.0465x; 1.0465x over previous
import functools

import jax
import jax.numpy as jnp
from jax import lax
from jax.experimental import pallas as pl
from jax.experimental.pallas import tpu as pltpu
from jax.experimental.pallas import tpu_sc as plsc

B, Kp, C, M = 2, 256, 512, 8
Kc = Kp * M
NBLK = 2
PB = Kp // NBLK
RB = PB * M
GRID = B * NBLK
PHI_M2 = 1.6 ** (-2)
JITTER = 1e-4
BETA = 0.5
K_TARGET = 1024.0
F32 = jnp.float32

NC, NS, L = 2, 16, 16
NW = NC * NS
RPW = (B * Kp) // NW
NCHUNK = Kp // L


def _sigmoid(x):
    return 1.0 / (1.0 + jnp.exp(-x))


def _softplus(x):
    return jnp.maximum(x, 0.0) + jnp.log(1.0 + jnp.exp(-jnp.abs(x)))


def _ln(x, g, b, eps=1e-5):
    m = jnp.mean(x, axis=-1, keepdims=True)
    v = jnp.mean((x - m) ** 2, axis=-1, keepdims=True)
    return (x - m) / jnp.sqrt(v + eps) * g + b


def _inv6(s00, s01, s02, s11, s12, s22):
    c00 = s11 * s22 - s12 * s12
    c01 = s02 * s12 - s01 * s22
    c02 = s01 * s12 - s02 * s11
    c11 = s00 * s22 - s02 * s02
    c12 = s01 * s02 - s00 * s12
    c22 = s00 * s11 - s01 * s01
    rdet = 1.0 / (s00 * c00 + s01 * c01 + s02 * c02)
    return c00 * rdet, c01 * rdet, c02 * rdet, c11 * rdet, c12 * rdet, c22 * rdet


def _quad(p00, p01, p02, p11, p12, p22, dx, dy, dz):
    return (p00 * dx * dx + p11 * dy * dy + p22 * dz * dz
            + 2.0 * (p01 * dx * dy + p02 * dx * dz + p12 * dy * dz))


def _sc_body(muT_hbm, sigT_hbm, sp_hbm, n_out, snei_out,
             mvT, sgT, idxv, rows, nbuf, sem):
    wid = lax.axis_index("s") * NC + lax.axis_index("c")
    bno = wid // NS
    jbase = (wid - bno * NS) * RPW

    pltpu.sync_copy(muT_hbm.at[pl.ds(bno * 3, 3), :], mvT)
    pltpu.sync_copy(sigT_hbm.at[pl.ds(bno * 9, 9), :], sgT)

    lane = lax.iota(jnp.int32, L)
    nloc_v = jnp.zeros((L,), jnp.int32)

    for r in range(RPW):
        jloc = jbase + r
        sel = lane == r
        mjx = jnp.sum(jnp.where(sel, mvT[0, pl.ds(jbase, L)], 0.0))
        mjy = jnp.sum(jnp.where(sel, mvT[1, pl.ds(jbase, L)], 0.0))
        mjz = jnp.sum(jnp.where(sel, mvT[2, pl.ds(jbase, L)], 0.0))

        def chunk_body(c, carry, mjx=mjx, mjy=mjy, mjz=mjz, jloc=jloc):
            mx, mi = carry
            off = c * L
            s00 = sgT[0, pl.ds(off, L)]
            s01 = sgT[1, pl.ds(off, L)]
            s02 = sgT[2, pl.ds(off, L)]
            s11 = sgT[4, pl.ds(off, L)]
            s12 = sgT[5, pl.ds(off, L)]
            s22 = sgT[8, pl.ds(off, L)]
            p00, p01, p02, p11, p12, p22 = _inv6(s00, s01, s02, s11, s12, s22)
            dx = mjx - mvT[0, pl.ds(off, L)]
            dy = mjy - mvT[1, pl.ds(off, L)]
            dz = mjz - mvT[2, pl.ds(off, L)]
            S = -0.5 * _quad(p00, p01, p02, p11, p12, p22, dx, dy, dz)
            kidx = lane + off
            S = jnp.where(kidx == jloc, -1e30, S)
            cond = S > mx
            mx = jnp.where(cond, S, mx)
            mi = jnp.where(cond, kidx.astype(F32), mi)
            return mx, mi

        mx, mi = lax.fori_loop(
            0, NCHUNK, chunk_body,
            (jnp.full((L,), -3e38, F32), jnp.zeros((L,), F32)))
        mbest = jnp.max(mx)
        nf = jnp.min(jnp.where(mx == mbest, mi, 1e9))
        nloc_v = jnp.where(sel, nf.astype(jnp.int32), nloc_v)

    nbuf[...] = nloc_v
    pltpu.sync_copy(nbuf, n_out.at[pl.ds(wid * RPW, RPW)])
    idxv[...] = nloc_v + bno * Kp
    pltpu.async_copy(sp_hbm.at[idxv], rows, sem).wait()
    pltpu.sync_copy(rows, snei_out.at[pl.ds(wid * RPW, RPW), :])


def _sc_retrieve(muT, sigT9, sp):
    mesh = plsc.VectorSubcoreMesh(core_axis_name="c", subcore_axis_name="s",
                                  num_cores=NC, num_subcores=NS)
    kern = pl.kernel(
        _sc_body,
        out_type=[jax.ShapeDtypeStruct((B * Kp,), jnp.int32),
                  jax.ShapeDtypeStruct((B * Kp, C), F32)],
        mesh=mesh,
        scratch_types=[pltpu.VMEM((3, Kp), F32),
                       pltpu.VMEM((9, Kp), F32),
                       pltpu.VMEM((RPW,), jnp.int32),
                       pltpu.VMEM((RPW, C), F32),
                       pltpu.VMEM((RPW,), jnp.int32),
                       pltpu.SemaphoreType.DMA],
        compiler_params=pltpu.CompilerParams(use_tc_tiling_on_sc=False,
                                             needs_layout_passes=False),
    )
    return kern(muT.reshape(B * 3, Kp), sigT9.reshape(B * 9, Kp), sp)


def _body(spb, mub, muall, sig9b, sig9all, maskb, xib, emb, nblk, sneib,
          glg, glb, gW1, gb1, gW2c, gb2,
          plg, plb, pW1, pb1, pW2c, pb2, gmW, gmb,
          out_s, out_mu, out_sig, out_g, out_gsum, out_loss):
    gid = pl.program_id(0)
    bno = gid // NBLK

    spb = spb[...]
    mub = mub[...]
    muall = muall[...]
    sig9b = sig9b[...]
    sig9all = sig9all[...]
    maskb = maskb[...]
    xib = xib[...]
    emb = emb[...]
    nblk = nblk[...]
    sneib = sneib[...]
    glg = glg[...]
    glb = glb[...]
    gW1 = gW1[...]
    gb1 = gb1[...]
    gW2c = gW2c[...]
    gb2 = gb2[...]
    plg = plg[...]
    plb = plb[...]
    pW1 = pW1[...]
    pb1 = pb1[...]
    pW2c = pW2c[...]
    pb2 = pb2[...]
    gmW = gmW[...]
    gmb = gmb[...]

    colk_i = jax.lax.broadcasted_iota(jnp.int32, (PB, Kp), 1)
    onehot = jnp.where(colk_i == nblk, 1.0, 0.0).astype(F32)

    hi = jax.lax.Precision.HIGHEST
    mu_nei = jnp.dot(onehot, muall, preferred_element_type=F32, precision=hi)
    sig_nei = jnp.dot(onehot, sig9all, preferred_element_type=F32, precision=hi)
    s_nei = sneib

    sb = [sig9b[:, i:i + 1] for i in (0, 1, 2, 4, 5, 8)]
    Pj = _inv6(*sb)
    sn = [sig_nei[:, i:i + 1] for i in (0, 1, 2, 4, 5, 8)]
    Pn = _inv6(*sn)

    a00 = sb[0] + 1e-6
    a11 = sb[3] + 1e-6
    a22 = sb[5] + 1e-6
    l11 = jnp.sqrt(a00)
    l21 = sb[1] / l11
    l31 = sb[2] / l11
    l22 = jnp.sqrt(a11 - l21 * l21)
    l32 = (sb[4] - l31 * l21) / l22
    l33 = jnp.sqrt(a22 - l31 * l31 - l32 * l32)

    hp = _ln(spb, plg, plb)
    hp = jnp.dot(hp, pW1, preferred_element_type=F32) + pb1
    hp = hp * _sigmoid(hp)
    apre = jnp.dot(hp, pW2c, preferred_element_type=F32) + pb2
    log_a = jnp.log(_softplus(apre) + 1e-8)

    def exp8(v):
        return jnp.broadcast_to(v[:, None, :], (PB, M, v.shape[1])).reshape(RB, v.shape[1])

    X = (spb[:, None, :] + emb[None, :, :]).reshape(RB, C)
    h = _ln(X, glg, glb)
    h = jnp.dot(h, gW1, preferred_element_type=F32) + gb1
    h = h * _sigmoid(h)
    blog = jnp.dot(h, gW2c, preferred_element_type=F32) + gb2
    blog = blog + BETA * exp8(log_a)
    g = _sigmoid(blog) * exp8(maskb)

    xi_bias = jnp.dot(emb, gmW, preferred_element_type=F32, precision=hi) + gmb
    xi = xib + jnp.broadcast_to(xi_bias[None, :, :], (PB, M, 3)).reshape(RB, 3)
    x1 = xi[:, 0:1]
    x2 = xi[:, 1:2]
    x3 = xi[:, 2:3]
    Le = exp8(jnp.concatenate([l11, l21, l22, l31, l32, l33], axis=1))
    mue = exp8(mub)
    mu0x = mue[:, 0:1] + Le[:, 0:1] * x1
    mu0y = mue[:, 1:2] + Le[:, 1:2] * x1 + Le[:, 2:3] * x2
    mu0z = mue[:, 2:3] + Le[:, 3:4] * x1 + Le[:, 4:5] * x2 + Le[:, 5:6] * x3

    Pje = exp8(jnp.concatenate(Pj, axis=1))
    Pne = exp8(jnp.concatenate(Pn, axis=1))
    mune = exp8(mu_nei)
    d0x = mu0x - mue[:, 0:1]
    d0y = mu0y - mue[:, 1:2]
    d0z = mu0z - mue[:, 2:3]
    d1x = mu0x - mune[:, 0:1]
    d1y = mu0y - mune[:, 1:2]
    d1z = mu0z - mune[:, 2:3]
    sc0 = -0.5 * _quad(Pje[:, 0:1], Pje[:, 1:2], Pje[:, 2:3],
                       Pje[:, 3:4], Pje[:, 4:5], Pje[:, 5:6], d0x, d0y, d0z)
    sc1 = -0.5 * _quad(Pne[:, 0:1], Pne[:, 1:2], Pne[:, 2:3],
                       Pne[:, 3:4], Pne[:, 4:5], Pne[:, 5:6], d1x, d1y, d1z)
    w0 = _sigmoid(sc0 - sc1)
    w1 = 1.0 - w0

    mcx = w0 * mue[:, 0:1] + w1 * mune[:, 0:1]
    mcy = w0 * mue[:, 1:2] + w1 * mune[:, 1:2]
    mcz = w0 * mue[:, 2:3] + w1 * mune[:, 2:3]
    out_mu[:, 0:1] = mcx
    out_mu[:, 1:2] = mcy
    out_mu[:, 2:3] = mcz

    sig6e = exp8(jnp.concatenate(sb, axis=1))
    sig6ne = exp8(jnp.concatenate(sn, axis=1))
    djx = mue[:, 0:1] - mcx
    djy = mue[:, 1:2] - mcy
    djz = mue[:, 2:3] - mcz
    dnx = mune[:, 0:1] - mcx
    dny = mune[:, 1:2] - mcy
    dnz = mune[:, 2:3] - mcz
    intra = (w0 * sig6e + w1 * sig6ne) * PHI_M2
    o00 = intra[:, 0:1] + w0 * djx * djx + w1 * dnx * dnx + JITTER
    o01 = intra[:, 1:2] + w0 * djx * djy + w1 * dnx * dny
    o02 = intra[:, 2:3] + w0 * djx * djz + w1 * dnx * dnz
    o11 = intra[:, 3:4] + w0 * djy * djy + w1 * dny * dny + JITTER
    o12 = intra[:, 4:5] + w0 * djy * djz + w1 * dny * dnz
    o22 = intra[:, 5:6] + w0 * djz * djz + w1 * dnz * dnz + JITTER
    out_sig[:, 0:1] = o00
    out_sig[:, 1:2] = o01
    out_sig[:, 2:3] = o02
    out_sig[:, 3:4] = o01
    out_sig[:, 4:5] = o11
    out_sig[:, 5:6] = o12
    out_sig[:, 6:7] = o02
    out_sig[:, 7:8] = o12
    out_sig[:, 8:9] = o22

    smix = w0 * exp8(spb) + w1 * exp8(s_nei)
    out_s[...] = g * smix
    out_g[...] = g

    @pl.when(gid == 0)
    def _init():
        out_gsum[...] = jnp.zeros((B, 1), F32)

    rowb = jax.lax.broadcasted_iota(jnp.int32, (B, 1), 0)
    out_gsum[...] += jnp.where(rowb == bno, jnp.sum(g), 0.0)

    @pl.when(gid == GRID - 1)
    def _loss():
        d = out_gsum[...] - K_TARGET
        out_loss[...] = jnp.sum(d * d, axis=0, keepdims=True) * (1.0 / B)


def kernel(s_parent, mu_p, Sigma_p, mask_parent, xi_noise, child_embed,
           gate_ln_g, gate_ln_b, gate_W1, gate_b1, gate_W2, gate_b2,
           prior_ln_g, prior_ln_b, prior_W1, prior_b1, prior_W2, prior_b2,
           geom_W, geom_b):
    f = F32
    sp = s_parent.reshape(B * Kp, C)
    mu = mu_p.reshape(B * Kp, 3)
    sig9 = Sigma_p.reshape(B * Kp, 9)
    muT = jnp.transpose(mu_p, (0, 2, 1))
    sigT9 = jnp.transpose(Sigma_p.reshape(B, Kp, 9), (0, 2, 1))
    maskf = mask_parent.reshape(B * Kp, 1)
    xif = xi_noise.reshape(B * Kc, 3)

    n_idx, s_nei = _sc_retrieve(muT, sigT9, sp)
    n2d = n_idx.reshape(B * Kp, 1)

    def bs(shape, idx):
        return pl.BlockSpec(shape, idx)

    blkmap = lambda g: (g, 0)
    batmap = lambda g: (g // NBLK, 0)
    full = lambda g: (0, 0)

    in_specs = [
        bs((PB, C), blkmap),
        bs((PB, 3), blkmap),
        bs((Kp, 3), batmap),
        bs((PB, 9), blkmap),
        bs((Kp, 9), batmap),
        bs((PB, 1), blkmap),
        bs((RB, 3), blkmap),
        bs((M, C), full),
        bs((PB, 1), blkmap),
        bs((PB, C), blkmap),
        bs((1, C), full), bs((1, C), full), bs((C, C), full), bs((1, C), full),
        bs((C, 1), full), bs((1, 1), full),
        bs((1, C), full), bs((1, C), full), bs((C, C), full), bs((1, C), full),
        bs((C, 1), full), bs((1, 1), full),
        bs((C, 3), full), bs((1, 3), full),
    ]
    out_specs = [
        bs((RB, C), blkmap),
        bs((RB, 3), blkmap),
        bs((RB, 9), blkmap),
        bs((RB, 1), blkmap),
        bs((B, 1), full),
        bs((1, 1), full),
    ]
    out_shapes = [
        jax.ShapeDtypeStruct((B * Kc, C), f),
        jax.ShapeDtypeStruct((B * Kc, 3), f),
        jax.ShapeDtypeStruct((B * Kc, 9), f),
        jax.ShapeDtypeStruct((B * Kc, 1), f),
        jax.ShapeDtypeStruct((B, 1), f),
        jax.ShapeDtypeStruct((1, 1), f),
    ]

    outs = pl.pallas_call(
        _body,
        grid=(GRID,),
        in_specs=in_specs,
        out_specs=out_specs,
        out_shape=out_shapes,
    )(sp, mu, mu, sig9, sig9, maskf, xif, child_embed, n2d, s_nei,
      gate_ln_g.reshape(1, C), gate_ln_b.reshape(1, C), gate_W1,
      gate_b1.reshape(1, C), gate_W2, gate_b2.reshape(1, 1),
      prior_ln_g.reshape(1, C), prior_ln_b.reshape(1, C), prior_W1,
      prior_b1.reshape(1, C), prior_W2, prior_b2.reshape(1, 1),
      geom_W, geom_b.reshape(1, 3))

    s_child, mu_child, sig9o, gout, _gsum, loss = outs
    return (s_child.reshape(B, Kc, C),
            mu_child.reshape(B, Kc, 3),
            sig9o.reshape(B, Kc, 3, 3),
            gout.reshape(B, Kc),
            loss.reshape(()))


# speedup vs baseline: 1.2377x; 1.1827x over previous
import jax
import jax.numpy as jnp
from jax.experimental import pallas as pl

B, Kp, C, M = 2, 256, 512, 8
Kc = Kp * M
NBLK = 4
PB = Kp // NBLK
RB = PB * M
GRID = B * NBLK
PHI_M2 = 1.6 ** (-2)
JITTER = 1e-4
BETA = 0.5
K_TARGET = 1024.0
F32 = jnp.float32


def _sigmoid(x):
    return 1.0 / (1.0 + jnp.exp(-x))


def _softplus(x):
    return jnp.maximum(x, 0.0) + jnp.log(1.0 + jnp.exp(-jnp.abs(x)))


def _ln(x, g, b, eps=1e-5):
    m = jnp.mean(x, axis=-1, keepdims=True)
    v = jnp.mean((x - m) ** 2, axis=-1, keepdims=True)
    return (x - m) / jnp.sqrt(v + eps) * g + b


def _inv6(s00, s01, s02, s11, s12, s22):
    c00 = s11 * s22 - s12 * s12
    c01 = s02 * s12 - s01 * s22
    c02 = s01 * s12 - s02 * s11
    c11 = s00 * s22 - s02 * s02
    c12 = s01 * s02 - s00 * s12
    c22 = s00 * s11 - s01 * s01
    rdet = 1.0 / (s00 * c00 + s01 * c01 + s02 * c02)
    return c00 * rdet, c01 * rdet, c02 * rdet, c11 * rdet, c12 * rdet, c22 * rdet


def _quad(p00, p01, p02, p11, p12, p22, dx, dy, dz):
    return (p00 * dx * dx + p11 * dy * dy + p22 * dz * dz
            + 2.0 * (p01 * dx * dy + p02 * dx * dz + p12 * dy * dz))


def _body(spb, spall, mub, muall, sig9b, sig9all, muT, sigT9, maskb, xib, emb,
          glg, glb, gW1, gb1, gW2c, gb2,
          plg, plb, pW1, pb1, pW2c, pb2, gmW, gmb,
          out_s, out_mu, out_sig, out_g, out_gsum, out_loss):
    gid = pl.program_id(0)
    bno = gid // NBLK
    blk = gid - bno * NBLK

    spb = spb[...]
    spall = spall[...]
    mub = mub[...]
    muall = muall[...]
    sig9b = sig9b[...]
    sig9all = sig9all[...]
    muT = muT[0]
    sigT9 = sigT9[0]
    maskb = maskb[...]
    xib = xib[...]
    emb = emb[...]
    glg = glg[...]
    glb = glb[...]
    gW1 = gW1[...]
    gb1 = gb1[...]
    gW2c = gW2c[...]
    gb2 = gb2[...]
    plg = plg[...]
    plb = plb[...]
    pW1 = pW1[...]
    pb1 = pb1[...]
    pW2c = pW2c[...]
    pb2 = pb2[...]
    gmW = gmW[...]
    gmb = gmb[...]

    t00 = sigT9[0:1, :]
    t01 = sigT9[1:2, :]
    t02 = sigT9[2:3, :]
    t11 = sigT9[4:5, :]
    t12 = sigT9[5:6, :]
    t22 = sigT9[8:9, :]
    p00, p01, p02, p11, p12, p22 = _inv6(t00, t01, t02, t11, t12, t22)
    x = muT[0:1, :]
    y = muT[1:2, :]
    z = muT[2:3, :]
    bx = mub[:, 0:1]
    by = mub[:, 1:2]
    bz = mub[:, 2:3]
    dx = bx - x
    dy = by - y
    dz = bz - z
    S = -0.5 * _quad(p00, p01, p02, p11, p12, p22, dx, dy, dz)

    rowj_i = jax.lax.broadcasted_iota(jnp.int32, (PB, Kp), 0) + blk * PB
    colk_i = jax.lax.broadcasted_iota(jnp.int32, (PB, Kp), 1)
    colk = colk_i.astype(F32)
    S = jnp.where(rowj_i == colk_i, -1e30, S)
    mmax = jnp.max(S, axis=1, keepdims=True)
    nf = jnp.min(jnp.where(S == mmax, colk, float(Kp)), axis=1, keepdims=True)
    onehot = jnp.where(colk == nf, 1.0, 0.0)

    s_nei = jnp.dot(onehot, spall, preferred_element_type=F32, precision=jax.lax.Precision.HIGHEST)
    mu_nei = jnp.dot(onehot, muall, preferred_element_type=F32, precision=jax.lax.Precision.HIGHEST)
    sig_nei = jnp.dot(onehot, sig9all, preferred_element_type=F32, precision=jax.lax.Precision.HIGHEST)

    sb = [sig9b[:, i:i + 1] for i in (0, 1, 2, 4, 5, 8)]
    Pj = _inv6(*sb)
    sn = [sig_nei[:, i:i + 1] for i in (0, 1, 2, 4, 5, 8)]
    Pn = _inv6(*sn)

    a00 = sb[0] + 1e-6
    a11 = sb[3] + 1e-6
    a22 = sb[5] + 1e-6
    l11 = jnp.sqrt(a00)
    l21 = sb[1] / l11
    l31 = sb[2] / l11
    l22 = jnp.sqrt(a11 - l21 * l21)
    l32 = (sb[4] - l31 * l21) / l22
    l33 = jnp.sqrt(a22 - l31 * l31 - l32 * l32)

    hp = _ln(spb, plg, plb)
    hp = jnp.dot(hp, pW1, preferred_element_type=F32) + pb1
    hp = hp * _sigmoid(hp)
    apre = jnp.dot(hp, pW2c, preferred_element_type=F32) + pb2
    log_a = jnp.log(_softplus(apre) + 1e-8)

    def exp8(v):
        return jnp.broadcast_to(v[:, None, :], (PB, M, v.shape[1])).reshape(RB, v.shape[1])

    X = (spb[:, None, :] + emb[None, :, :]).reshape(RB, C)
    h = _ln(X, glg, glb)
    h = jnp.dot(h, gW1, preferred_element_type=F32) + gb1
    h = h * _sigmoid(h)
    blog = jnp.dot(h, gW2c, preferred_element_type=F32) + gb2
    blog = blog + BETA * exp8(log_a)
    g = _sigmoid(blog) * exp8(maskb)

    xi_bias = jnp.dot(emb, gmW, preferred_element_type=F32, precision=jax.lax.Precision.HIGHEST) + gmb
    xi = xib + jnp.broadcast_to(xi_bias[None, :, :], (PB, M, 3)).reshape(RB, 3)
    x1 = xi[:, 0:1]
    x2 = xi[:, 1:2]
    x3 = xi[:, 2:3]
    Le = exp8(jnp.concatenate([l11, l21, l22, l31, l32, l33], axis=1))
    mue = exp8(mub)
    mu0x = mue[:, 0:1] + Le[:, 0:1] * x1
    mu0y = mue[:, 1:2] + Le[:, 1:2] * x1 + Le[:, 2:3] * x2
    mu0z = mue[:, 2:3] + Le[:, 3:4] * x1 + Le[:, 4:5] * x2 + Le[:, 5:6] * x3

    Pje = exp8(jnp.concatenate(Pj, axis=1))
    Pne = exp8(jnp.concatenate(Pn, axis=1))
    mune = exp8(mu_nei)
    d0x = mu0x - mue[:, 0:1]
    d0y = mu0y - mue[:, 1:2]
    d0z = mu0z - mue[:, 2:3]
    d1x = mu0x - mune[:, 0:1]
    d1y = mu0y - mune[:, 1:2]
    d1z = mu0z - mune[:, 2:3]
    sc0 = -0.5 * _quad(Pje[:, 0:1], Pje[:, 1:2], Pje[:, 2:3],
                       Pje[:, 3:4], Pje[:, 4:5], Pje[:, 5:6], d0x, d0y, d0z)
    sc1 = -0.5 * _quad(Pne[:, 0:1], Pne[:, 1:2], Pne[:, 2:3],
                       Pne[:, 3:4], Pne[:, 4:5], Pne[:, 5:6], d1x, d1y, d1z)
    w0 = _sigmoid(sc0 - sc1)
    w1 = 1.0 - w0

    mcx = w0 * mue[:, 0:1] + w1 * mune[:, 0:1]
    mcy = w0 * mue[:, 1:2] + w1 * mune[:, 1:2]
    mcz = w0 * mue[:, 2:3] + w1 * mune[:, 2:3]
    out_mu[:, 0:1] = mcx
    out_mu[:, 1:2] = mcy
    out_mu[:, 2:3] = mcz

    sig6e = exp8(jnp.concatenate(sb, axis=1))
    sig6ne = exp8(jnp.concatenate(sn, axis=1))
    djx = mue[:, 0:1] - mcx
    djy = mue[:, 1:2] - mcy
    djz = mue[:, 2:3] - mcz
    dnx = mune[:, 0:1] - mcx
    dny = mune[:, 1:2] - mcy
    dnz = mune[:, 2:3] - mcz
    intra = (w0 * sig6e + w1 * sig6ne) * PHI_M2
    o00 = intra[:, 0:1] + w0 * djx * djx + w1 * dnx * dnx + JITTER
    o01 = intra[:, 1:2] + w0 * djx * djy + w1 * dnx * dny
    o02 = intra[:, 2:3] + w0 * djx * djz + w1 * dnx * dnz
    o11 = intra[:, 3:4] + w0 * djy * djy + w1 * dny * dny + JITTER
    o12 = intra[:, 4:5] + w0 * djy * djz + w1 * dny * dnz
    o22 = intra[:, 5:6] + w0 * djz * djz + w1 * dnz * dnz + JITTER
    out_sig[:, 0:1] = o00
    out_sig[:, 1:2] = o01
    out_sig[:, 2:3] = o02
    out_sig[:, 3:4] = o01
    out_sig[:, 4:5] = o11
    out_sig[:, 5:6] = o12
    out_sig[:, 6:7] = o02
    out_sig[:, 7:8] = o12
    out_sig[:, 8:9] = o22

    smix = w0 * exp8(spb) + w1 * exp8(s_nei)
    out_s[...] = g * smix
    out_g[...] = g

    @pl.when(gid == 0)
    def _init():
        out_gsum[...] = jnp.zeros((B, 1), F32)

    rowb = jax.lax.broadcasted_iota(jnp.int32, (B, 1), 0)
    out_gsum[...] += jnp.where(rowb == bno, jnp.sum(g), 0.0)

    @pl.when(gid == GRID - 1)
    def _loss():
        d = out_gsum[...] - K_TARGET
        out_loss[...] = jnp.sum(d * d, axis=0, keepdims=True) * (1.0 / B)


def kernel(s_parent, mu_p, Sigma_p, mask_parent, xi_noise, child_embed,
           gate_ln_g, gate_ln_b, gate_W1, gate_b1, gate_W2, gate_b2,
           prior_ln_g, prior_ln_b, prior_W1, prior_b1, prior_W2, prior_b2,
           geom_W, geom_b):
    f = F32
    sp = s_parent.reshape(B * Kp, C)
    mu = mu_p.reshape(B * Kp, 3)
    sig9 = Sigma_p.reshape(B * Kp, 9)
    muT = jnp.transpose(mu_p, (0, 2, 1))
    sigT9 = jnp.transpose(Sigma_p.reshape(B, Kp, 9), (0, 2, 1))
    maskf = mask_parent.reshape(B * Kp, 1)
    xif = xi_noise.reshape(B * Kc, 3)

    def bs(shape, idx):
        return pl.BlockSpec(shape, idx)

    blkmap = lambda g: (g, 0)
    batmap = lambda g: (g // NBLK, 0)
    full = lambda g: (0, 0)

    in_specs = [
        bs((PB, C), blkmap),
        bs((Kp, C), batmap),
        bs((PB, 3), blkmap),
        bs((Kp, 3), batmap),
        bs((PB, 9), blkmap),
        bs((Kp, 9), batmap),
        pl.BlockSpec((1, 3, Kp), lambda g: (g // NBLK, 0, 0)),
        pl.BlockSpec((1, 9, Kp), lambda g: (g // NBLK, 0, 0)),
        bs((PB, 1), blkmap),
        bs((RB, 3), blkmap),
        bs((M, C), full),
        bs((1, C), full), bs((1, C), full), bs((C, C), full), bs((1, C), full),
        bs((C, 1), full), bs((1, 1), full),
        bs((1, C), full), bs((1, C), full), bs((C, C), full), bs((1, C), full),
        bs((C, 1), full), bs((1, 1), full),
        bs((C, 3), full), bs((1, 3), full),
    ]
    out_specs = [
        bs((RB, C), blkmap),
        bs((RB, 3), blkmap),
        bs((RB, 9), blkmap),
        bs((RB, 1), blkmap),
        bs((B, 1), full),
        bs((1, 1), full),
    ]
    out_shapes = [
        jax.ShapeDtypeStruct((B * Kc, C), f),
        jax.ShapeDtypeStruct((B * Kc, 3), f),
        jax.ShapeDtypeStruct((B * Kc, 9), f),
        jax.ShapeDtypeStruct((B * Kc, 1), f),
        jax.ShapeDtypeStruct((B, 1), f),
        jax.ShapeDtypeStruct((1, 1), f),
    ]

    outs = pl.pallas_call(
        _body,
        grid=(GRID,),
        in_specs=in_specs,
        out_specs=out_specs,
        out_shape=out_shapes,
    )(sp, sp, mu, mu, sig9, sig9, muT, sigT9, maskf, xif, child_embed,
      gate_ln_g.reshape(1, C), gate_ln_b.reshape(1, C), gate_W1,
      gate_b1.reshape(1, C), gate_W2, gate_b2.reshape(1, 1),
      prior_ln_g.reshape(1, C), prior_ln_b.reshape(1, C), prior_W1,
      prior_b1.reshape(1, C), prior_W2, prior_b2.reshape(1, 1),
      geom_W, geom_b.reshape(1, 3))

    s_child, mu_child, sig9o, gout, _gsum, loss = outs
    return (s_child.reshape(B, Kc, C),
            mu_child.reshape(B, Kc, 3),
            sig9o.reshape(B, Kc, 3, 3),
            gout.reshape(B, Kc),
            loss.reshape(()))
